# Initial kernel scaffold; baseline (speedup 1.0000x reference)
import jax
import jax.numpy as jnp
from jax.experimental import pallas as pl


def kernel(q_lat, q_lon, kv_lat, kv_lon, attention_span):
    raise NotImplementedError("write your pallas kernel here")


# speedup vs baseline: 2.2653x; 2.2653x over previous
import jax
import jax.numpy as jnp
from jax.experimental import pallas as pl

EARTH_RADIUS_KM = 6371.0


def _tile_kernel(span_ref, qlat_ref, qlon_ref, kvlat_ref, kvlon_ref,
                 mask_ref, dist_ref):
    qlat = qlat_ref[...]
    qlon = qlon_ref[...]
    kvlat = kvlat_ref[...]
    kvlon = kvlon_ref[...]
    span = span_ref[0, 0]

    sl1 = jnp.sin(0.5 * qlat)
    cl1 = jnp.cos(0.5 * qlat)
    so1 = jnp.sin(0.5 * qlon)
    co1 = jnp.cos(0.5 * qlon)
    sl2 = jnp.sin(0.5 * kvlat)
    cl2 = jnp.cos(0.5 * kvlat)
    so2 = jnp.sin(0.5 * kvlon)
    co2 = jnp.cos(0.5 * kvlon)
    coslat1 = 1.0 - 2.0 * sl1 * sl1
    coslat2 = 1.0 - 2.0 * sl2 * sl2

    s = sl2 * cl1 - cl2 * sl1
    t = so2 * co1 - co2 * so1
    a = s * s + (coslat1 * coslat2) * (t * t)
    c = 2.0 * jnp.arctan2(jnp.sqrt(a), jnp.sqrt(1.0 - a))
    dist = EARTH_RADIUS_KM * c
    dist_ref[...] = dist
    mask_ref[...] = dist <= span


def kernel(q_lat, q_lon, kv_lat, kv_lon, attention_span):
    Q = q_lat.shape[0]
    KV = kv_lat.shape[0]
    BQ = 512
    BK = 2048
    q_lat2 = q_lat.reshape(Q, 1)
    q_lon2 = q_lon.reshape(Q, 1)
    kv_lat2 = kv_lat.reshape(1, KV)
    kv_lon2 = kv_lon.reshape(1, KV)
    span = jnp.asarray(attention_span, jnp.float32).reshape(1, 1)

    grid = (Q // BQ, KV // BK)
    span_spec = pl.BlockSpec((1, 1), lambda i, j: (0, 0))
    q_spec = pl.BlockSpec((BQ, 1), lambda i, j: (i, 0))
    kv_spec = pl.BlockSpec((1, BK), lambda i, j: (0, j))
    out_spec = pl.BlockSpec((BQ, BK), lambda i, j: (i, j))

    mask, dist = pl.pallas_call(
        _tile_kernel,
        grid=grid,
        in_specs=[span_spec, q_spec, q_spec, kv_spec, kv_spec],
        out_specs=[out_spec, out_spec],
        out_shape=[
            jax.ShapeDtypeStruct((Q, KV), jnp.bool_),
            jax.ShapeDtypeStruct((Q, KV), jnp.float32),
        ],
    )(span, q_lat2, q_lon2, kv_lat2, kv_lon2)
    return mask, dist
